# Initial kernel scaffold; baseline (speedup 1.0000x reference)
import jax
import jax.numpy as jnp
from jax.experimental import pallas as pl


def kernel(x, edge_index, node_types, hyperedge_types, hyperedge_attr, W_lin, b_lin, ln_g, ln_b, W_sheaf, b_sheaf, W_c1, b_c1, W_c2, b_c2):
    raise NotImplementedError("write your pallas kernel here")


# speedup vs baseline: 7.3873x; 7.3873x over previous
import functools

import jax
import jax.numpy as jnp
from jax import lax
from jax.experimental import pallas as pl
from jax.experimental.pallas import tpu as pltpu
from jax.experimental.pallas import tpu_sc as plsc

F32 = jnp.float32
I32 = jnp.int32

D = 6
HID = 32
N = 10000
E = 160000
F = D * HID
HW = 128

NC = 2
NS = 16
NW = NC * NS
CH = 128
NCHUNKS = E // CH
BASE_CHUNKS = NCHUNKS // NW
EXTRA = NCHUNKS % NW
NP = 10240
RPT = NP // NS

_mesh = plsc.VectorSubcoreMesh(
    core_axis_name="c", subcore_axis_name="s", num_cores=NC, num_subcores=NS)


def _worker(c, s):
    wid = s * NC + c
    nch = BASE_CHUNKS + jnp.where(wid < EXTRA, 1, 0)
    cbase = wid * BASE_CHUNKS + jnp.minimum(wid, EXTRA)
    return wid, nch, cbase


def _rsqrt(w):
    i = lax.bitcast_convert_type(w, I32)
    i = 0x5F3759DF - lax.shift_right_logical(i, 1)
    y = lax.bitcast_convert_type(i, F32)
    for _ in range(3):
        y = y * (1.5 - 0.5 * w * y * y)
    return y


def _splat_i32(v):
    return jnp.full((16,), v, I32)


@functools.partial(
    pl.kernel,
    out_type=jax.ShapeDtypeStruct((E, 16), F32),
    mesh=_mesh,
    scratch_types=(
        pltpu.VMEM((CH,), I32),
        pltpu.VMEM((CH,), I32),
        pltpu.VMEM((CH, HW), F32),
        pltpu.VMEM((CH, HW), F32),
        pltpu.VMEM((CH, 16), F32),
        pltpu.VMEM((16,), F32),
        pltpu.SemaphoreType.DMA,
        pltpu.SemaphoreType.DMA,
    ),
    compiler_params=pltpu.CompilerParams(needs_layout_passes=False),
)
def _sc_alpha(row_h, col_h, an_h, ae_h, bc_h,
              alpha_h,
              rb, cb, ab, eb, al, bcv, sem0, sem1):
    c = lax.axis_index("c")
    s = lax.axis_index("s")
    wid, nch, cbase = _worker(c, s)

    def _arow(i, carry):
        al[i, :] = jnp.zeros((16,), F32)
        return carry
    lax.fori_loop(0, CH, _arow, 0)
    pltpu.sync_copy(bc_h, bcv)

    def _chunk(j, carry):
        off = (cbase + j) * CH
        pltpu.sync_copy(row_h.at[pl.ds(off, CH)], rb)
        pltpu.sync_copy(col_h.at[pl.ds(off, CH)], cb)
        pltpu.async_copy(an_h.at[rb], ab, sem0).wait()
        pltpu.async_copy(ae_h.at[cb], eb, sem1).wait()
        c6 = _splat_i32(6)
        c7 = _splat_i32(7)
        for g in range(CH // 16):
            ei = lax.iota(I32, 16) + g * 16
            s1 = plsc.load_gather(ab, [ei, c6])
            s2 = plsc.load_gather(ab, [ei, c7])
            t1 = plsc.load_gather(eb, [ei, c6])
            t2 = plsc.load_gather(eb, [ei, c7])
            mu = (s1 + t1) * (1.0 / (2 * HID))
            ex2 = (s2 + t2) * (1.0 / (2 * HID))
            inv = _rsqrt(ex2 - mu * mu + 1e-5)
            for k in range(D):
                ck = _splat_i32(k)
                pk = plsc.load_gather(ab, [ei, ck])
                qk = plsc.load_gather(eb, [ei, ck])
                bk = plsc.load_gather(bcv, [ck])
                z = (pk + qk) * inv + bk
                a = 1.0 / (1.0 + jnp.exp(-z))
                plsc.store_scatter(al, [ei, ck], a)
        pltpu.sync_copy(al, alpha_h.at[pl.ds(off, CH)])
        return carry
    lax.fori_loop(0, nch, _chunk, 0)


ZR = 64


@functools.partial(
    pl.kernel,
    out_type=jax.ShapeDtypeStruct((2, NC, NP, HW), F32),
    mesh=_mesh,
    scratch_types=(
        pltpu.VMEM_SHARED((NP, HW), F32),
        pltpu.VMEM((ZR, HW), F32),
        pltpu.VMEM((CH,), I32),
        pltpu.VMEM((CH,), I32),
        pltpu.VMEM((CH, 16), F32),
        pltpu.VMEM((CH, HW), F32),
        pltpu.SemaphoreType.DMA,
    ),
    compiler_params=pltpu.CompilerParams(needs_layout_passes=False),
)
def _sc_pass(t0_h, t1_h, src_h, dst_h, alpha_h,
             outp_h,
             acc, zb, sb, db, ab, gb, sem):
    c = lax.axis_index("c")
    s = lax.axis_index("s")
    wid, nch, cbase = _worker(c, s)

    def _zrow(i, carry):
        for v in range(HW // 16):
            zb[i, pl.ds(16 * v, 16)] = jnp.zeros((16,), F32)
        return carry
    lax.fori_loop(0, ZR, _zrow, 0)

    for h, t_h in ((0, t0_h), (1, t1_h)):
        for q in range(RPT // ZR):
            pltpu.sync_copy(zb, acc.at[pl.ds(s * RPT + q * ZR, ZR)])
        plsc.subcore_barrier()

        def _chunk(j, carry):
            off = (cbase + j) * CH
            pltpu.sync_copy(src_h.at[pl.ds(off, CH)], sb)
            pltpu.sync_copy(dst_h.at[pl.ds(off, CH)], db)
            pltpu.sync_copy(alpha_h.at[pl.ds(off, CH)], ab)
            pltpu.async_copy(t_h.at[sb], gb, sem).wait()

            if h == 0:
                def _edge(e, ecarry):
                    e16 = jnp.full((16,), e, I32)
                    for k in range(4):
                        av = plsc.load_gather(ab, [e16, _splat_i32(k)])
                        for v in (2 * k, 2 * k + 1):
                            gb[e, pl.ds(16 * v, 16)] = (
                                gb[e, pl.ds(16 * v, 16)] * av)
                    return ecarry
            else:
                def _edge(e, ecarry):
                    e16 = jnp.full((16,), e, I32)
                    for k in (4, 5):
                        av = plsc.load_gather(ab, [e16, _splat_i32(k)])
                        for v in (2 * (k - 4), 2 * (k - 4) + 1):
                            gb[e, pl.ds(16 * v, 16)] = (
                                gb[e, pl.ds(16 * v, 16)] * av)
                    gb[e, pl.ds(64, 16)] = (
                        gb[e, pl.ds(64, 16)] * ab[e, pl.ds(0, 16)])
                    return ecarry
            lax.fori_loop(0, CH, _edge, 0, unroll=4)
            pltpu.sync_copy(gb, acc.at[db], add=True)
            return carry
        lax.fori_loop(0, nch, _chunk, 0)
        plsc.subcore_barrier()
        pltpu.sync_copy(acc.at[pl.ds(s * RPT, RPT)],
                        outp_h.at[h, c, pl.ds(s * RPT, RPT)])


BN = 1000


def _split_halves(M, m0_ref, m1_ref):
    m0_ref[...] = M[:, :HW]
    m1_ref[...] = jnp.concatenate(
        [M[:, HW:], jnp.full((BN, 16), 1.0, F32), jnp.zeros((BN, 48), F32)],
        axis=1)


def _prep_body(x_ref, ea_ref, wl_ref, bl_ref, wc1_ref, bc1_ref, m_ref,
               gxwx_ref, gewe_ref, g64_ref,
               t0_ref, t1_ref, an_ref, ae_ref):
    def tables(src):
        h0 = jnp.dot(src, wl_ref[...], preferred_element_type=F32) + bl_ref[...]
        xp = jnp.dot(h0, m_ref[...], preferred_element_type=F32)
        s1 = jnp.sum(xp, axis=1, keepdims=True)
        s2 = jnp.sum(xp * xp, axis=1, keepdims=True)
        return h0, xp, s1, s2

    h0, xp, s1, s2 = tables(x_ref[...])
    h1 = jnp.dot(h0, wc1_ref[...], preferred_element_type=F32) + bc1_ref[...]
    _split_halves(h1, t0_ref, t1_ref)
    p = jnp.dot(xp, gxwx_ref[...], preferred_element_type=F32) - s1 * g64_ref[...]
    an_ref[...] = jnp.concatenate(
        [p, s1, s2, jnp.zeros((BN, HW - 8), F32)], axis=1)

    _, ep, t1s, t2s = tables(ea_ref[...])
    q = jnp.dot(ep, gewe_ref[...], preferred_element_type=F32) - t1s * g64_ref[...]
    ae_ref[...] = jnp.concatenate(
        [q, t1s, t2s, jnp.zeros((BN, HW - 8), F32)], axis=1)


def _tc_prep(x, ea, wl, bl, wc1bd, bc1r, m, gxwx, gewe, g64):
    full = lambda shape: pl.BlockSpec(shape, lambda i: tuple(0 for _ in shape))
    return pl.pallas_call(
        _prep_body,
        grid=(N // BN,),
        in_specs=[
            pl.BlockSpec((BN, 128), lambda i: (i, 0)),
            pl.BlockSpec((BN, 128), lambda i: (i, 0)),
            full((128, F)), full((1, F)), full((F, F)), full((1, F)),
            full((F, HID)), full((HID, D)), full((HID, D)), full((1, D)),
        ],
        out_specs=[
            pl.BlockSpec((BN, HW), lambda i: (i, 0)),
            pl.BlockSpec((BN, HW), lambda i: (i, 0)),
            pl.BlockSpec((BN, HW), lambda i: (i, 0)),
            pl.BlockSpec((BN, HW), lambda i: (i, 0)),
        ],
        out_shape=[
            jax.ShapeDtypeStruct((N, HW), F32),
            jax.ShapeDtypeStruct((N, HW), F32),
            jax.ShapeDtypeStruct((N, HW), F32),
            jax.ShapeDtypeStruct((N, HW), F32),
        ],
    )(x, ea, wl, bl, wc1bd, bc1r, m, gxwx, gewe, g64)


def _normed(p_ref, s_ref):
    P0 = p_ref[0, 0] + p_ref[0, 1]
    P1 = p_ref[1, 0] + p_ref[1, 1]
    d = P1[:, 64:80]
    dinv = jnp.where(d > 0, 1.0 / d, 0.0)
    scale = jnp.dot(dinv, s_ref[...], preferred_element_type=F32)
    return jnp.concatenate([P0, P1[:, :64]], axis=1) * scale


def _elu(h):
    return jnp.where(h > 0, h, jnp.exp(jnp.minimum(h, 0.0)) - 1.0)


def _comb_mid_body(p_ref, s_ref, m0_ref, m1_ref):
    _split_halves(_normed(p_ref, s_ref), m0_ref, m1_ref)


def _comb_conv_body(p_ref, s_ref, w_ref, b_ref, m0_ref, m1_ref):
    h = _elu(_normed(p_ref, s_ref))
    h2 = jnp.dot(h, w_ref[...], preferred_element_type=F32) + b_ref[...]
    _split_halves(h2, m0_ref, m1_ref)


def _comb_out_body(p_ref, s_ref, o_ref):
    o_ref[...] = _elu(_normed(p_ref, s_ref))


def _comb_specs():
    return [
        pl.BlockSpec((2, 2, BN, HW), lambda i: (0, 0, i, 0)),
        pl.BlockSpec((16, F), lambda i: (0, 0)),
    ]


def _half_out():
    return (
        [pl.BlockSpec((BN, HW), lambda i: (i, 0)),
         pl.BlockSpec((BN, HW), lambda i: (i, 0))],
        [jax.ShapeDtypeStruct((N, HW), F32),
         jax.ShapeDtypeStruct((N, HW), F32)],
    )


def _comb_mid(p, sel):
    specs, shapes = _half_out()
    return pl.pallas_call(
        _comb_mid_body, grid=(N // BN,),
        in_specs=_comb_specs(),
        out_specs=specs, out_shape=shapes,
    )(p, sel)


def _comb_conv(p, sel, w, b):
    specs, shapes = _half_out()
    return pl.pallas_call(
        _comb_conv_body, grid=(N // BN,),
        in_specs=_comb_specs() + [
            pl.BlockSpec((F, F), lambda i: (0, 0)),
            pl.BlockSpec((1, F), lambda i: (0, 0)),
        ],
        out_specs=specs, out_shape=shapes,
    )(p, sel, w, b)


def _comb_out(p, sel):
    return pl.pallas_call(
        _comb_out_body, grid=(N // BN,),
        in_specs=_comb_specs(),
        out_specs=pl.BlockSpec((BN, F), lambda i: (i, 0)),
        out_shape=jax.ShapeDtypeStruct((N, F), F32),
    )(p, sel)


def kernel(x, edge_index, node_types, hyperedge_types, hyperedge_attr,
           W_lin, b_lin, ln_g, ln_b, W_sheaf, b_sheaf, W_c1, b_c1, W_c2, b_c2):
    row = edge_index[0]
    col = edge_index[1]

    eye = jnp.eye(D, dtype=F32)
    wc1bd = jnp.kron(eye, W_c1)
    wc2bd = jnp.kron(eye, W_c2)
    bc1r = jnp.tile(b_c1, D)[None, :]
    bc2r = jnp.tile(b_c2, D)[None, :]
    m = jnp.kron(jnp.ones((D, 1), F32), jnp.eye(HID, dtype=F32)) / D
    gx, ge = ln_g[:HID], ln_g[HID:]
    wx, we = W_sheaf[:HID], W_sheaf[HID:]
    gxwx = gx[:, None] * wx
    gewe = ge[:, None] * we
    g = ln_g @ W_sheaf
    g64 = (g / (2 * HID))[None, :]
    bconst = jnp.zeros((16,), F32).at[:D].set(b_sheaf + ln_b @ W_sheaf)
    sel = (jnp.arange(F)[None, :] // HID == jnp.arange(16)[:, None]).astype(F32)

    t0, t1, an, ae = _tc_prep(x, hyperedge_attr, W_lin, b_lin[None, :], wc1bd,
                              bc1r, m, gxwx, gewe, g64)

    alpha = _sc_alpha(row, col, an, ae, bconst)

    p = _sc_pass(t0, t1, row, col, alpha)
    m0, m1 = _comb_mid(p, sel)
    p = _sc_pass(m0, m1, col, row, alpha)
    h20, h21 = _comb_conv(p, sel, wc2bd, bc2r)
    p = _sc_pass(h20, h21, row, col, alpha)
    n0, n1 = _comb_mid(p, sel)
    p = _sc_pass(n0, n1, col, row, alpha)
    return _comb_out(p, sel)


# speedup vs baseline: 8.9902x; 1.2170x over previous
import functools

import jax
import jax.numpy as jnp
from jax import lax
from jax.experimental import pallas as pl
from jax.experimental.pallas import tpu as pltpu
from jax.experimental.pallas import tpu_sc as plsc

F32 = jnp.float32
I32 = jnp.int32

D = 6
HID = 32
N = 10000
E = 160000
F = D * HID
HW = 128

NC = 2
NS = 16
NW = NC * NS
CH = 128
NCHUNKS = E // CH
BASE_CHUNKS = NCHUNKS // NW
EXTRA = NCHUNKS % NW
NP = 10112
RPT = NP // NS

_mesh = plsc.VectorSubcoreMesh(
    core_axis_name="c", subcore_axis_name="s", num_cores=NC, num_subcores=NS)


def _worker(c, s):
    wid = s * NC + c
    nch = BASE_CHUNKS + jnp.where(wid < EXTRA, 1, 0)
    cbase = wid * BASE_CHUNKS + jnp.minimum(wid, EXTRA)
    return wid, nch, cbase


def _rsqrt(w):
    i = lax.bitcast_convert_type(w, I32)
    i = 0x5F3759DF - lax.shift_right_logical(i, 1)
    y = lax.bitcast_convert_type(i, F32)
    for _ in range(3):
        y = y * (1.5 - 0.5 * w * y * y)
    return y


def _splat_i32(v):
    return jnp.full((16,), v, I32)


@functools.partial(
    pl.kernel,
    out_type=jax.ShapeDtypeStruct((E, 16), F32),
    mesh=_mesh,
    scratch_types=(
        pltpu.VMEM((2, CH), I32),
        pltpu.VMEM((CH, HW), F32),
        pltpu.VMEM((CH, HW), F32),
        pltpu.VMEM((CH, 16), F32),
        pltpu.VMEM((16,), F32),
        pltpu.SemaphoreType.DMA,
        pltpu.SemaphoreType.DMA,
    ),
    compiler_params=pltpu.CompilerParams(needs_layout_passes=False),
)
def _sc_alpha(idx_h, an_h, ae_h, bc_h,
              alpha_h,
              ib, ab, eb, al, bcv, sem0, sem1):
    c = lax.axis_index("c")
    s = lax.axis_index("s")
    wid, nch, cbase = _worker(c, s)

    def _arow(i, carry):
        al[i, :] = jnp.zeros((16,), F32)
        return carry
    lax.fori_loop(0, CH, _arow, 0)
    pltpu.sync_copy(bc_h, bcv)

    def _chunk(j, carry):
        off = (cbase + j) * CH
        pltpu.sync_copy(idx_h.at[cbase + j], ib)
        g0 = pltpu.async_copy(an_h.at[ib.at[0]], ab, sem0)
        g1 = pltpu.async_copy(ae_h.at[ib.at[1]], eb, sem1)
        g0.wait()
        g1.wait()
        c6 = _splat_i32(6)
        c7 = _splat_i32(7)
        for g in range(CH // 16):
            ei = lax.iota(I32, 16) + g * 16
            s1 = plsc.load_gather(ab, [ei, c6])
            s2 = plsc.load_gather(ab, [ei, c7])
            t1 = plsc.load_gather(eb, [ei, c6])
            t2 = plsc.load_gather(eb, [ei, c7])
            mu = (s1 + t1) * (1.0 / (2 * HID))
            ex2 = (s2 + t2) * (1.0 / (2 * HID))
            inv = _rsqrt(ex2 - mu * mu + 1e-5)
            for k in range(D):
                ck = _splat_i32(k)
                pk = plsc.load_gather(ab, [ei, ck])
                qk = plsc.load_gather(eb, [ei, ck])
                bk = plsc.load_gather(bcv, [ck])
                z = (pk + qk) * inv + bk
                a = 1.0 / (1.0 + jnp.exp(-z))
                plsc.store_scatter(al, [ei, ck], a)
        pltpu.sync_copy(al, alpha_h.at[pl.ds(off, CH)])
        return carry
    lax.fori_loop(0, nch, _chunk, 0)


ZR = 16


def _scale_edges(gb, ab, h):
    if h == 0:
        def _edge(e, ecarry):
            e16 = jnp.full((16,), e, I32)
            for k in range(4):
                av = plsc.load_gather(ab, [e16, _splat_i32(k)])
                for v in (2 * k, 2 * k + 1):
                    gb[e, pl.ds(16 * v, 16)] = gb[e, pl.ds(16 * v, 16)] * av
            return ecarry
    else:
        def _edge(e, ecarry):
            e16 = jnp.full((16,), e, I32)
            for k in (4, 5):
                av = plsc.load_gather(ab, [e16, _splat_i32(k)])
                for v in (2 * (k - 4), 2 * (k - 4) + 1):
                    gb[e, pl.ds(16 * v, 16)] = gb[e, pl.ds(16 * v, 16)] * av
            gb[e, pl.ds(64, 16)] = (
                gb[e, pl.ds(64, 16)] * ab[e, pl.ds(0, 16)])
            return ecarry
    lax.fori_loop(0, CH, _edge, 0, unroll=4)


@functools.partial(
    pl.kernel,
    out_type=jax.ShapeDtypeStruct((2, NC, NP, HW), F32),
    mesh=_mesh,
    scratch_types=(
        pltpu.VMEM_SHARED((NP, HW), F32),
        pltpu.VMEM((2, CH), I32),
        pltpu.VMEM((2, CH), I32),
        pltpu.VMEM((CH, 16), F32),
        pltpu.VMEM((CH, HW), F32),
        pltpu.VMEM((CH, HW), F32),
        pltpu.SemaphoreType.DMA,
        pltpu.SemaphoreType.DMA,
        pltpu.SemaphoreType.DMA,
        pltpu.SemaphoreType.DMA,
    ),
    compiler_params=pltpu.CompilerParams(needs_layout_passes=False),
)
def _sc_pass(t0_h, t1_h, idx_h, alpha_h,
             outp_h,
             acc, ib0, ib1, ab, gb0, gb1,
             gsem0, gsem1, ssem0, ssem1):
    c = lax.axis_index("c")
    s = lax.axis_index("s")
    wid, nch, cbase = _worker(c, s)
    npairs = nch // 2
    ztail = RPT - (RPT // ZR) * ZR

    for h, t_h in ((0, t0_h), (1, t1_h)):
        def _zrow(i, carry):
            for v in range(HW // 16):
                gb0[i, pl.ds(16 * v, 16)] = jnp.zeros((16,), F32)
            return carry
        lax.fori_loop(0, ZR, _zrow, 0)
        for q in range(RPT // ZR):
            pltpu.sync_copy(gb0.at[pl.ds(0, ZR)],
                            acc.at[pl.ds(s * RPT + q * ZR, ZR)])
        if ztail:
            pltpu.sync_copy(gb0.at[pl.ds(0, ztail)],
                            acc.at[pl.ds(s * RPT + (RPT // ZR) * ZR, ztail)])
        plsc.subcore_barrier()

        def _pair(i, carry):
            c0 = cbase + 2 * i
            c1 = c0 + 1
            pltpu.sync_copy(idx_h.at[c0], ib0)
            pltpu.sync_copy(idx_h.at[c1], ib1)
            g0 = pltpu.async_copy(t_h.at[ib0.at[0]], gb0, gsem0)
            g1 = pltpu.async_copy(t_h.at[ib1.at[0]], gb1, gsem1)
            pltpu.sync_copy(alpha_h.at[pl.ds(c0 * CH, CH)], ab)
            g0.wait()
            _scale_edges(gb0, ab, h)
            s0 = pltpu.async_copy(gb0, acc.at[ib0.at[1]], ssem0, add=True)
            pltpu.sync_copy(alpha_h.at[pl.ds(c1 * CH, CH)], ab)
            g1.wait()
            _scale_edges(gb1, ab, h)
            s0.wait()
            s1 = pltpu.async_copy(gb1, acc.at[ib1.at[1]], ssem1, add=True)
            s1.wait()
            return carry
        lax.fori_loop(0, npairs, _pair, 0)

        @pl.when(nch > 2 * npairs)
        def _tail():
            ct = cbase + 2 * npairs
            pltpu.sync_copy(idx_h.at[ct], ib0)
            pltpu.sync_copy(alpha_h.at[pl.ds(ct * CH, CH)], ab)
            pltpu.async_copy(t_h.at[ib0.at[0]], gb0, gsem0).wait()
            _scale_edges(gb0, ab, h)
            pltpu.sync_copy(gb0, acc.at[ib0.at[1]], add=True)

        plsc.subcore_barrier()
        pltpu.sync_copy(acc.at[pl.ds(s * RPT, RPT)],
                        outp_h.at[h, c, pl.ds(s * RPT, RPT)])


BN = 1000


def _split_halves(M, m0_ref, m1_ref):
    m0_ref[...] = M[:, :HW]
    m1_ref[...] = jnp.concatenate(
        [M[:, HW:], jnp.full((BN, 16), 1.0, F32), jnp.zeros((BN, 48), F32)],
        axis=1)


def _prep_body(x_ref, ea_ref, wl_ref, bl_ref, wc1_ref, bc1_ref, m_ref,
               gxwx_ref, gewe_ref, g64_ref,
               t0_ref, t1_ref, an_ref, ae_ref):
    def tables(src):
        h0 = jnp.dot(src, wl_ref[...], preferred_element_type=F32) + bl_ref[...]
        xp = jnp.dot(h0, m_ref[...], preferred_element_type=F32)
        s1 = jnp.sum(xp, axis=1, keepdims=True)
        s2 = jnp.sum(xp * xp, axis=1, keepdims=True)
        return h0, xp, s1, s2

    h0, xp, s1, s2 = tables(x_ref[...])
    h1 = jnp.dot(h0, wc1_ref[...], preferred_element_type=F32) + bc1_ref[...]
    _split_halves(h1, t0_ref, t1_ref)
    p = jnp.dot(xp, gxwx_ref[...], preferred_element_type=F32) - s1 * g64_ref[...]
    an_ref[...] = jnp.concatenate(
        [p, s1, s2, jnp.zeros((BN, HW - 8), F32)], axis=1)

    _, ep, t1s, t2s = tables(ea_ref[...])
    q = jnp.dot(ep, gewe_ref[...], preferred_element_type=F32) - t1s * g64_ref[...]
    ae_ref[...] = jnp.concatenate(
        [q, t1s, t2s, jnp.zeros((BN, HW - 8), F32)], axis=1)


def _tc_prep(x, ea, wl, bl, wc1bd, bc1r, m, gxwx, gewe, g64):
    full = lambda shape: pl.BlockSpec(shape, lambda i: tuple(0 for _ in shape))
    return pl.pallas_call(
        _prep_body,
        grid=(N // BN,),
        in_specs=[
            pl.BlockSpec((BN, 128), lambda i: (i, 0)),
            pl.BlockSpec((BN, 128), lambda i: (i, 0)),
            full((128, F)), full((1, F)), full((F, F)), full((1, F)),
            full((F, HID)), full((HID, D)), full((HID, D)), full((1, D)),
        ],
        out_specs=[
            pl.BlockSpec((BN, HW), lambda i: (i, 0)),
            pl.BlockSpec((BN, HW), lambda i: (i, 0)),
            pl.BlockSpec((BN, HW), lambda i: (i, 0)),
            pl.BlockSpec((BN, HW), lambda i: (i, 0)),
        ],
        out_shape=[
            jax.ShapeDtypeStruct((N, HW), F32),
            jax.ShapeDtypeStruct((N, HW), F32),
            jax.ShapeDtypeStruct((N, HW), F32),
            jax.ShapeDtypeStruct((N, HW), F32),
        ],
    )(x, ea, wl, bl, wc1bd, bc1r, m, gxwx, gewe, g64)


def _normed(p_ref, s_ref):
    P0 = p_ref[0, 0] + p_ref[0, 1]
    P1 = p_ref[1, 0] + p_ref[1, 1]
    d = P1[:, 64:80]
    dinv = jnp.where(d > 0, 1.0 / d, 0.0)
    scale = jnp.dot(dinv, s_ref[...], preferred_element_type=F32)
    return jnp.concatenate([P0, P1[:, :64]], axis=1) * scale


def _elu(h):
    return jnp.where(h > 0, h, jnp.exp(jnp.minimum(h, 0.0)) - 1.0)


def _comb_mid_body(p_ref, s_ref, m0_ref, m1_ref):
    _split_halves(_normed(p_ref, s_ref), m0_ref, m1_ref)


def _comb_conv_body(p_ref, s_ref, w_ref, b_ref, m0_ref, m1_ref):
    h = _elu(_normed(p_ref, s_ref))
    h2 = jnp.dot(h, w_ref[...], preferred_element_type=F32) + b_ref[...]
    _split_halves(h2, m0_ref, m1_ref)


def _comb_out_body(p_ref, s_ref, o_ref):
    o_ref[...] = _elu(_normed(p_ref, s_ref))


def _comb_specs():
    return [
        pl.BlockSpec((2, 2, BN, HW), lambda i: (0, 0, i, 0)),
        pl.BlockSpec((16, F), lambda i: (0, 0)),
    ]


def _half_out():
    return (
        [pl.BlockSpec((BN, HW), lambda i: (i, 0)),
         pl.BlockSpec((BN, HW), lambda i: (i, 0))],
        [jax.ShapeDtypeStruct((N, HW), F32),
         jax.ShapeDtypeStruct((N, HW), F32)],
    )


def _comb_mid(p, sel):
    specs, shapes = _half_out()
    return pl.pallas_call(
        _comb_mid_body, grid=(N // BN,),
        in_specs=_comb_specs(),
        out_specs=specs, out_shape=shapes,
    )(p, sel)


def _comb_conv(p, sel, w, b):
    specs, shapes = _half_out()
    return pl.pallas_call(
        _comb_conv_body, grid=(N // BN,),
        in_specs=_comb_specs() + [
            pl.BlockSpec((F, F), lambda i: (0, 0)),
            pl.BlockSpec((1, F), lambda i: (0, 0)),
        ],
        out_specs=specs, out_shape=shapes,
    )(p, sel, w, b)


def _comb_out(p, sel):
    return pl.pallas_call(
        _comb_out_body, grid=(N // BN,),
        in_specs=_comb_specs(),
        out_specs=pl.BlockSpec((BN, F), lambda i: (i, 0)),
        out_shape=jax.ShapeDtypeStruct((N, F), F32),
    )(p, sel)


def kernel(x, edge_index, node_types, hyperedge_types, hyperedge_attr,
           W_lin, b_lin, ln_g, ln_b, W_sheaf, b_sheaf, W_c1, b_c1, W_c2, b_c2):
    row = edge_index[0]
    col = edge_index[1]

    eye = jnp.eye(D, dtype=F32)
    wc1bd = jnp.kron(eye, W_c1)
    wc2bd = jnp.kron(eye, W_c2)
    bc1r = jnp.tile(b_c1, D)[None, :]
    bc2r = jnp.tile(b_c2, D)[None, :]
    m = jnp.kron(jnp.ones((D, 1), F32), jnp.eye(HID, dtype=F32)) / D
    gx, ge = ln_g[:HID], ln_g[HID:]
    wx, we = W_sheaf[:HID], W_sheaf[HID:]
    gxwx = gx[:, None] * wx
    gewe = ge[:, None] * we
    g = ln_g @ W_sheaf
    g64 = (g / (2 * HID))[None, :]
    bconst = jnp.zeros((16,), F32).at[:D].set(b_sheaf + ln_b @ W_sheaf)
    sel = (jnp.arange(F)[None, :] // HID == jnp.arange(16)[:, None]).astype(F32)

    rowc = row.reshape(NCHUNKS, CH)
    colc = col.reshape(NCHUNKS, CH)
    idx_rc = jnp.stack([rowc, colc], axis=1)
    idx_cr = jnp.stack([colc, rowc], axis=1)

    t0, t1, an, ae = _tc_prep(x, hyperedge_attr, W_lin, b_lin[None, :], wc1bd,
                              bc1r, m, gxwx, gewe, g64)

    alpha = _sc_alpha(idx_rc, an, ae, bconst)

    p = _sc_pass(t0, t1, idx_rc, alpha)
    m0, m1 = _comb_mid(p, sel)
    p = _sc_pass(m0, m1, idx_cr, alpha)
    h20, h21 = _comb_conv(p, sel, wc2bd, bc2r)
    p = _sc_pass(h20, h21, idx_rc, alpha)
    n0, n1 = _comb_mid(p, sel)
    p = _sc_pass(n0, n1, idx_cr, alpha)
    return _comb_out(p, sel)
